# baseline (device time: 20860 ns/iter reference)
import jax
import jax.numpy as jnp
from jax import lax
from jax.experimental import pallas as pl
from jax.experimental.pallas import tpu as pltpu

N_DEV = 4
B, SQ, SKV, H_LOC, DH = 2, 128, 128, 4, 64
D_MODEL = 512
D_CTX = H_LOC * DH
BLK = 64


def _body(x_ref, wq_ref, k_ref, v_ref, wo_ref, out_ref,
          local_ref, comm_ref, send_sems, recv_sems):
    my = lax.axis_index("i")

    qb = lax.broadcasted_iota(jnp.int32, (SQ, SKV), 0) // BLK
    kb = lax.broadcasted_iota(jnp.int32, (SQ, SKV), 1) // BLK
    mask = kb <= qb

    wq_loc = wq_ref[:, pl.ds(my * D_CTX, D_CTX)]
    for b in range(B):
        xb = x_ref[b]
        q_all = jnp.dot(xb, wq_loc, preferred_element_type=jnp.float32)
        ctx_parts = []
        for h in range(H_LOC):
            q_h = q_all[:, h * DH:(h + 1) * DH]
            k_h = k_ref[b, :, h, :]
            v_h = v_ref[b, :, h, :]
            s = lax.dot_general(
                q_h, k_h, (((1,), (1,)), ((), ())),
                preferred_element_type=jnp.float32,
            ) * 0.125
            s = jnp.where(mask, s, -1e9)
            m = jnp.max(s, axis=1, keepdims=True)
            w = jnp.exp(s - m)
            w = w / jnp.sum(w, axis=1, keepdims=True)
            ctx_parts.append(
                jnp.dot(w, v_h, preferred_element_type=jnp.float32))
        local_ref[b] = jnp.concatenate(ctx_parts, axis=1)

    barrier_sem = pltpu.get_barrier_semaphore()
    for r in range(1, N_DEV):
        pl.semaphore_signal(
            barrier_sem, inc=1,
            device_id=((my + r) % N_DEV,),
            device_id_type=pl.DeviceIdType.MESH,
        )
    pl.semaphore_wait(barrier_sem, N_DEV - 1)

    def wo_block(origin):
        return wo_ref[pl.ds(origin * D_CTX, D_CTX), :]

    def add_contrib(src, origin, first=False):
        for b in range(B):
            c = jnp.dot(src[b], wo_block(origin),
                        preferred_element_type=jnp.float32)
            if first:
                out_ref[b] = c
            else:
                out_ref[b] = out_ref[b] + c

    rdmas = []
    for r in range(1, N_DEV):
        rdmas.append(pltpu.make_async_remote_copy(
            src_ref=local_ref,
            dst_ref=comm_ref.at[r - 1],
            send_sem=send_sems.at[r - 1],
            recv_sem=recv_sems.at[r - 1],
            device_id=((my + r) % N_DEV,),
            device_id_type=pl.DeviceIdType.MESH,
        ))
    for rdma in rdmas:
        rdma.start()

    add_contrib(local_ref, my, first=True)

    for r in range(1, N_DEV):
        rdmas[r - 1].wait_recv()
        add_contrib(comm_ref.at[r - 1], (my - r) % N_DEV)

    for rdma in rdmas:
        rdma.wait_send()


def kernel(x, Wq, K_ext, V_ext, Wo):
    return pl.pallas_call(
        _body,
        out_shape=jax.ShapeDtypeStruct((B, SQ, D_MODEL), jnp.float32),
        in_specs=[pl.BlockSpec(memory_space=pltpu.VMEM)] * 5,
        out_specs=pl.BlockSpec(memory_space=pltpu.VMEM),
        scratch_shapes=[
            pltpu.VMEM((B, SQ, D_CTX), jnp.float32),
            pltpu.VMEM((N_DEV - 1, B, SQ, D_CTX), jnp.float32),
            pltpu.SemaphoreType.DMA((N_DEV - 1,)),
            pltpu.SemaphoreType.DMA((N_DEV - 1,)),
        ],
        compiler_params=pltpu.CompilerParams(collective_id=0),
    )(x, Wq, K_ext, V_ext, Wo)


# device time: 13424 ns/iter; 1.5539x vs baseline; 1.5539x over previous
import jax
import jax.numpy as jnp
from jax import lax
from jax.experimental import pallas as pl
from jax.experimental.pallas import tpu as pltpu

N_DEV = 4
B, SQ, SKV, H_LOC, DH = 2, 128, 128, 4, 64
D_MODEL = 512
D_CTX = H_LOC * DH
BLK = 64


def _body(x_ref, wq_ref, k_ref, v_ref, wo_ref, out_ref,
          local_ref, comm_ref, send_sems, recv_sems):
    my = lax.axis_index("i")

    barrier_sem = pltpu.get_barrier_semaphore()
    for r in range(1, N_DEV):
        pl.semaphore_signal(
            barrier_sem, inc=1,
            device_id=((my + r) % N_DEV,),
            device_id_type=pl.DeviceIdType.MESH,
        )
    pl.semaphore_wait(barrier_sem, N_DEV - 1)

    qb = lax.broadcasted_iota(jnp.int32, (SQ, SKV), 0) // BLK
    kb = lax.broadcasted_iota(jnp.int32, (SQ, SKV), 1) // BLK
    mask = kb <= qb

    rdmas = {}
    for b in range(B):
        q_all = jnp.dot(x_ref[b], wq_ref[...],
                        preferred_element_type=jnp.float32)
        ctx_parts = []
        for h in range(H_LOC):
            q_h = q_all[:, h * DH:(h + 1) * DH]
            k_h = k_ref[b * H_LOC + h]
            v_h = v_ref[b * H_LOC + h]
            s = lax.dot_general(
                q_h, k_h, (((1,), (1,)), ((), ())),
                preferred_element_type=jnp.float32,
            ) * 0.125
            s = jnp.where(mask, s, -1e9)
            m = jnp.max(s, axis=1, keepdims=True)
            w = jnp.exp(s - m)
            w = w / jnp.sum(w, axis=1, keepdims=True)
            ctx_parts.append(
                jnp.dot(w, v_h, preferred_element_type=jnp.float32))
        local_ref[b] = jnp.concatenate(ctx_parts, axis=1).astype(
            jnp.bfloat16)
        for r in range(1, N_DEV):
            rdma = pltpu.make_async_remote_copy(
                src_ref=local_ref.at[b],
                dst_ref=comm_ref.at[r - 1, b],
                send_sem=send_sems.at[r - 1, b],
                recv_sem=recv_sems.at[r - 1, b],
                device_id=((my + r) % N_DEV,),
                device_id_type=pl.DeviceIdType.MESH,
            )
            rdma.start()
            rdmas[(r, b)] = rdma

    def wo_block(origin):
        return wo_ref[pl.ds(origin * D_CTX, D_CTX), :]

    for b in range(B):
        out_ref[b] = jnp.dot(local_ref[b].astype(jnp.float32),
                             wo_block(my),
                             preferred_element_type=jnp.float32)

    for r in (1, 3, 2):
        for b in range(B):
            rdmas[(r, b)].wait_recv()
            out_ref[b] = out_ref[b] + jnp.dot(
                comm_ref[r - 1, b].astype(jnp.float32),
                wo_block((my - r) % N_DEV),
                preferred_element_type=jnp.float32)

    for rdma in rdmas.values():
        rdma.wait_send()


def kernel(x, Wq, K_ext, V_ext, Wo):
    my = lax.axis_index("i")
    wq_loc = lax.dynamic_slice_in_dim(Wq, my * D_CTX, D_CTX, axis=1)
    k_loc = K_ext.transpose(0, 2, 1, 3).reshape(B * H_LOC, SKV, DH)
    v_loc = V_ext.transpose(0, 2, 1, 3).reshape(B * H_LOC, SKV, DH)

    return pl.pallas_call(
        _body,
        out_shape=jax.ShapeDtypeStruct((B, SQ, D_MODEL), jnp.float32),
        in_specs=[pl.BlockSpec(memory_space=pltpu.VMEM)] * 5,
        out_specs=pl.BlockSpec(memory_space=pltpu.VMEM),
        scratch_shapes=[
            pltpu.VMEM((B, SQ, D_CTX), jnp.bfloat16),
            pltpu.VMEM((N_DEV - 1, B, SQ, D_CTX), jnp.bfloat16),
            pltpu.SemaphoreType.DMA((N_DEV - 1, B)),
            pltpu.SemaphoreType.DMA((N_DEV - 1, B)),
        ],
        compiler_params=pltpu.CompilerParams(collective_id=0),
    )(x, wq_loc, k_loc, v_loc, Wo)
